# Initial kernel scaffold; baseline (speedup 1.0000x reference)
import jax
import jax.numpy as jnp
from jax.experimental import pallas as pl


def kernel(batch, loc_repr_base):
    raise NotImplementedError("write your pallas kernel here")


# speedup vs baseline: 928.1261x; 928.1261x over previous
import functools

import jax
import jax.numpy as jnp
from jax import lax
from jax.experimental import pallas as pl
from jax.experimental.pallas import tpu as pltpu
from jax.experimental.pallas import tpu_sc as plsc

BINS = 128
CELLS = BINS * BINS
DELTA = 1.0 / BINS
NC = 2
NS = 16
NW = NC * NS
LANES = 16


def _sc_body(locs_hbm, default_hbm, out_hbm, w_ref, lbuf, linbuf):
    n_batch = out_hbm.shape[0]
    per_w = n_batch // NW
    lp = lbuf.shape[1]
    n_grp = lp // LANES
    wid = lax.axis_index("s") * NC + lax.axis_index("c")

    lane = lax.iota(jnp.int32, LANES)
    ch = lax.bitwise_and(lane, 3)
    mask4 = lane < 4
    is01 = ch < 2
    is2 = ch == 2
    ones = jnp.full((LANES,), 1.0, jnp.float32)
    corr0 = jnp.full((LANES,), 0.63, jnp.float32)

    gather_dnums = lax.GatherDimensionNumbers(
        offset_dims=(), collapsed_slice_dims=(0,), start_index_map=(0,)
    )

    def splat(v, l):
        idx = jnp.full((LANES, 1), l, jnp.int32)
        return lax.gather(
            v,
            idx,
            gather_dnums,
            slice_sizes=(1,),
            mode=lax.GatherScatterMode.PROMISE_IN_BOUNDS,
        )

    pltpu.sync_copy(default_hbm, w_ref)

    @pl.loop(0, per_w)
    def _sample(k):
        b = wid * per_w + k
        pltpu.sync_copy(locs_hbm.at[b], lbuf)

        @pl.loop(0, n_grp)
        def _scatter(g):
            xg = lbuf[0, pl.ds(g * LANES, LANES)]
            yg = lbuf[1, pl.ds(g * LANES, LANES)]
            xi = (xg * float(BINS)).astype(jnp.int32)
            yi = (yg * float(BINS)).astype(jnp.int32)
            lin = xi * BINS + yi
            linbuf[pl.ds(g * LANES, LANES)] = lin
            for l in range(LANES):
                linl = splat(lin, l)
                xl = splat(xg, l)
                yl = splat(yg, l)
                payload = jnp.where(is01, ones, jnp.where(is2, xl, yl))
                plsc.store_scatter(w_ref, [linl * 4 + ch], payload, mask=mask4)

        pltpu.sync_copy(w_ref, out_hbm.at[b])

        @pl.loop(0, n_grp)
        def _restore(g):
            lin = linbuf[pl.ds(g * LANES, LANES)]
            b2 = lax.bitwise_and(lin, BINS - 1).astype(jnp.float32) * DELTA
            b3 = lax.shift_right_logical(lin, 7).astype(jnp.float32) * DELTA
            for l in range(LANES):
                linl = splat(lin, l)
                b2l = splat(b2, l)
                b3l = splat(b3, l)
                payload = jnp.where(is01, corr0, jnp.where(is2, b2l, b3l))
                plsc.store_scatter(w_ref, [linl * 4 + ch], payload, mask=mask4)


@jax.jit
def kernel(batch, loc_repr_base):
    n_batch, n_loc, _ = batch.shape
    lp = ((n_loc + LANES - 1) // LANES) * LANES
    bt = jnp.swapaxes(batch, 1, 2)
    pad = jnp.broadcast_to(bt[:, :, n_loc - 1 :], (n_batch, 2, lp - n_loc))
    locs = jnp.concatenate([bt, pad], axis=2)
    default_cells = jnp.concatenate(
        [jnp.full((CELLS, 2), 0.63, jnp.float32), loc_repr_base], axis=1
    ).reshape(CELLS * 4)

    mesh = plsc.VectorSubcoreMesh(
        core_axis_name="c", subcore_axis_name="s", num_cores=NC, num_subcores=NS
    )
    out = pl.kernel(
        _sc_body,
        out_type=jax.ShapeDtypeStruct((n_batch, CELLS * 4), jnp.float32),
        mesh=mesh,
        scratch_types=[
            pltpu.VMEM((CELLS * 4,), jnp.float32),
            pltpu.VMEM((2, lp), jnp.float32),
            pltpu.VMEM((lp,), jnp.int32),
        ],
        compiler_params=pltpu.CompilerParams(needs_layout_passes=False),
    )(locs, default_cells)
    return out.reshape(n_batch, BINS, BINS, 4)


# speedup vs baseline: 1975.9658x; 2.1290x over previous
import functools

import jax
import jax.numpy as jnp
from jax import lax
from jax.experimental import pallas as pl
from jax.experimental.pallas import tpu as pltpu
from jax.experimental.pallas import tpu_sc as plsc

BINS = 128
CELLS = BINS * BINS
DELTA = 1.0 / BINS
NC = 2
NS = 16
NW = NC * NS
LANES = 16


def _sc_body(locs_hbm, default_hbm, out_hbm, w_ref, lbuf, linbuf):
    n_batch = out_hbm.shape[0]
    per_w = n_batch // NW
    lp = lbuf.shape[1]
    n_grp = lp // LANES
    wid = lax.axis_index("s") * NC + lax.axis_index("c")

    lane = lax.iota(jnp.int32, LANES)
    ch = lax.bitwise_and(lane, 3)
    ch128 = lax.shift_left(ch, 7)
    mask4 = lane < 4
    is01 = ch < 2
    is2 = ch == 2
    ones = jnp.full((LANES,), 1.0, jnp.float32)
    corr0 = jnp.full((LANES,), 0.63, jnp.float32)

    gather_dnums = lax.GatherDimensionNumbers(
        offset_dims=(), collapsed_slice_dims=(0,), start_index_map=(0,)
    )

    def splat(v, l):
        idx = jnp.full((LANES, 1), l, jnp.int32)
        return lax.gather(
            v,
            idx,
            gather_dnums,
            slice_sizes=(1,),
            mode=lax.GatherScatterMode.PROMISE_IN_BOUNDS,
        )

    pltpu.sync_copy(default_hbm, w_ref)

    @pl.loop(0, per_w)
    def _sample(k):
        b = wid * per_w + k
        pltpu.sync_copy(locs_hbm.at[b], lbuf)

        @pl.loop(0, n_grp)
        def _scatter(g):
            xg = lbuf[0, pl.ds(g * LANES, LANES)]
            yg = lbuf[1, pl.ds(g * LANES, LANES)]
            xi = (xg * float(BINS)).astype(jnp.int32)
            yi = (yg * float(BINS)).astype(jnp.int32)
            a = xi * (4 * BINS) + yi
            linbuf[pl.ds(g * LANES, LANES)] = a
            for l in range(LANES):
                al = splat(a, l)
                xl = splat(xg, l)
                yl = splat(yg, l)
                payload = jnp.where(is01, ones, jnp.where(is2, xl, yl))
                plsc.store_scatter(w_ref, [al + ch128], payload, mask=mask4)

        pltpu.sync_copy(w_ref, out_hbm.at[b])

        @pl.loop(0, n_grp)
        def _restore(g):
            a = linbuf[pl.ds(g * LANES, LANES)]
            b2 = lax.bitwise_and(a, BINS - 1).astype(jnp.float32) * DELTA
            b3 = lax.shift_right_logical(a, 9).astype(jnp.float32) * DELTA
            for l in range(LANES):
                al = splat(a, l)
                b2l = splat(b2, l)
                b3l = splat(b3, l)
                payload = jnp.where(is01, corr0, jnp.where(is2, b2l, b3l))
                plsc.store_scatter(w_ref, [al + ch128], payload, mask=mask4)


@jax.jit
def kernel(batch, loc_repr_base):
    n_batch, n_loc, _ = batch.shape
    lp = ((n_loc + LANES - 1) // LANES) * LANES
    bt = jnp.swapaxes(batch, 1, 2)
    pad = jnp.broadcast_to(bt[:, :, n_loc - 1 :], (n_batch, 2, lp - n_loc))
    locs = jnp.concatenate([bt, pad], axis=2)
    default_cells = (
        jnp.concatenate(
            [jnp.full((CELLS, 2), 0.63, jnp.float32), loc_repr_base], axis=1
        )
        .reshape(BINS, BINS, 4)
        .transpose(0, 2, 1)
        .reshape(CELLS * 4)
    )

    mesh = plsc.VectorSubcoreMesh(
        core_axis_name="c", subcore_axis_name="s", num_cores=NC, num_subcores=NS
    )
    out = pl.kernel(
        _sc_body,
        out_type=jax.ShapeDtypeStruct((n_batch, CELLS * 4), jnp.float32),
        mesh=mesh,
        scratch_types=[
            pltpu.VMEM((CELLS * 4,), jnp.float32),
            pltpu.VMEM((2, lp), jnp.float32),
            pltpu.VMEM((lp,), jnp.int32),
        ],
        compiler_params=pltpu.CompilerParams(needs_layout_passes=False),
    )(locs, default_cells)
    out = out.reshape(n_batch, BINS, 4, BINS)
    return out.transpose(0, 1, 3, 2)


# speedup vs baseline: 5169.6504x; 2.6163x over previous
import functools

import jax
import jax.numpy as jnp
from jax import lax
from jax.experimental import pallas as pl
from jax.experimental.pallas import tpu as pltpu
from jax.experimental.pallas import tpu_sc as plsc

BINS = 128
CELLS = BINS * BINS
DELTA = 1.0 / BINS
NC = 2
NS = 16
NW = NC * NS
LANES = 16


def _sc_body(locs_hbm, default_hbm, out_hbm, w_ref, lbuf, linbuf):
    n_batch = out_hbm.shape[0] // (CELLS * 4)
    per_w = n_batch // NW
    lp = lbuf.shape[1]
    n_grp = lp // LANES
    wid = lax.axis_index("s") * NC + lax.axis_index("c")

    lane = lax.iota(jnp.int32, LANES)
    ch = lax.bitwise_and(lane, 3)
    ch128 = lax.shift_left(ch, 7)
    mask4 = lane < 4
    is01 = ch < 2
    is2 = ch == 2
    ones = jnp.full((LANES,), 1.0, jnp.float32)
    corr0 = jnp.full((LANES,), 0.63, jnp.float32)

    gather_dnums = lax.GatherDimensionNumbers(
        offset_dims=(), collapsed_slice_dims=(0,), start_index_map=(0,)
    )

    def splat(v, l):
        idx = jnp.full((LANES, 1), l, jnp.int32)
        return lax.gather(
            v,
            idx,
            gather_dnums,
            slice_sizes=(1,),
            mode=lax.GatherScatterMode.PROMISE_IN_BOUNDS,
        )

    pltpu.sync_copy(default_hbm, w_ref)

    @pl.loop(0, per_w)
    def _sample(k):
        b = wid * per_w + k
        pltpu.sync_copy(locs_hbm.at[b], lbuf)
        out_slot = out_hbm.at[pl.ds(b * (CELLS * 4), CELLS * 4)]

        @pl.loop(0, n_grp)
        def _scatter(g):
            xg = lbuf[0, pl.ds(g * LANES, LANES)]
            yg = lbuf[1, pl.ds(g * LANES, LANES)]
            xi = (xg * float(BINS)).astype(jnp.int32)
            yi = (yg * float(BINS)).astype(jnp.int32)
            a = xi * (4 * BINS) + yi
            linbuf[pl.ds(g * LANES, LANES)] = a
            for l in range(LANES):
                al = splat(a, l)
                xl = splat(xg, l)
                yl = splat(yg, l)
                payload = jnp.where(is01, ones, jnp.where(is2, xl, yl))
                plsc.store_scatter(w_ref, [al + ch128], payload, mask=mask4)

        pltpu.sync_copy(w_ref, out_slot)

        @pl.loop(0, n_grp)
        def _restore(g):
            a = linbuf[pl.ds(g * LANES, LANES)]
            b2 = lax.bitwise_and(a, BINS - 1).astype(jnp.float32) * DELTA
            b3 = lax.shift_right_logical(a, 9).astype(jnp.float32) * DELTA
            for l in range(LANES):
                al = splat(a, l)
                b2l = splat(b2, l)
                b3l = splat(b3, l)
                payload = jnp.where(is01, corr0, jnp.where(is2, b2l, b3l))
                plsc.store_scatter(w_ref, [al + ch128], payload, mask=mask4)


@jax.jit
def kernel(batch, loc_repr_base):
    n_batch, n_loc, _ = batch.shape
    lp = ((n_loc + LANES - 1) // LANES) * LANES
    bt = jnp.swapaxes(batch, 1, 2)
    pad = jnp.broadcast_to(bt[:, :, n_loc - 1 :], (n_batch, 2, lp - n_loc))
    locs = jnp.concatenate([bt, pad], axis=2)
    default_cells = (
        jnp.concatenate(
            [jnp.full((CELLS, 2), 0.63, jnp.float32), loc_repr_base], axis=1
        )
        .reshape(BINS, BINS, 4)
        .transpose(0, 2, 1)
        .reshape(CELLS * 4)
    )

    mesh = plsc.VectorSubcoreMesh(
        core_axis_name="c", subcore_axis_name="s", num_cores=NC, num_subcores=NS
    )
    out = pl.kernel(
        _sc_body,
        out_type=jax.ShapeDtypeStruct((n_batch * CELLS * 4,), jnp.float32),
        mesh=mesh,
        scratch_types=[
            pltpu.VMEM((CELLS * 4,), jnp.float32),
            pltpu.VMEM((2, lp), jnp.float32),
            pltpu.VMEM((lp,), jnp.int32),
        ],
        compiler_params=pltpu.CompilerParams(needs_layout_passes=False),
    )(locs, default_cells)
    out = out.reshape(n_batch, BINS, 4, BINS)
    return out.transpose(0, 1, 3, 2)
